# baseline (speedup 1.0000x reference)
import functools

import jax
import jax.numpy as jnp
from jax import lax
from jax.experimental import pallas as pl
from jax.experimental.pallas import tpu as pltpu
from jax.experimental.pallas import tpu_sc as plsc

BATCH = 16384
EMBED = 32
HIDDEN = 256
OUT = 10

NC = 2
NS = 16
NW = NC * NS
BPW = BATCH // NW
CHUNK = 128
NCH = BPW // CHUNK


def _gather_body(svc_hbm, loc_hbm, tim_hbm, ts_hbm, tl_hbm, tt_hbm,
                 out_s, out_l, out_t,
                 idx_v, rows_s, rows_l, rows_t, sem):
    wid = lax.axis_index("s") * NC + lax.axis_index("c")
    base = wid * BPW
    descs = []
    tables = ((svc_hbm, ts_hbm, rows_s),
              (loc_hbm, tl_hbm, rows_l),
              (tim_hbm, tt_hbm, rows_t))
    for t, (ih, th, rv) in enumerate(tables):
        for ci in range(NCH):
            pltpu.sync_copy(ih.at[pl.ds(base + ci * CHUNK, CHUNK)],
                            idx_v.at[t, ci])
            descs.append(
                pltpu.async_copy(th.at[idx_v.at[t, ci]],
                                 rv.at[pl.ds(ci * CHUNK, CHUNK)], sem))
    for d in descs:
        d.wait()
    pltpu.sync_copy(rows_s, out_s.at[pl.ds(base, BPW)])
    pltpu.sync_copy(rows_l, out_l.at[pl.ds(base, BPW)])
    pltpu.sync_copy(rows_t, out_t.at[pl.ds(base, BPW)])


_sc_gather = functools.partial(
    pl.kernel,
    out_type=[jax.ShapeDtypeStruct((BATCH, EMBED), jnp.float32)] * 3,
    mesh=plsc.VectorSubcoreMesh(core_axis_name="c", subcore_axis_name="s"),
    scratch_types=[
        pltpu.VMEM((3, NCH, CHUNK), jnp.int32),
        pltpu.VMEM((BPW, EMBED), jnp.float32),
        pltpu.VMEM((BPW, EMBED), jnp.float32),
        pltpu.VMEM((BPW, EMBED), jnp.float32),
        pltpu.SemaphoreType.DMA,
    ],
    compiler_params=pltpu.CompilerParams(use_tc_tiling_on_sc=False),
)(_gather_body)


TILE = 2048


def _mlp_body(xs, xl, xt, w1, b1, w2, b2, w3, b3, out):
    x = jnp.concatenate([xs[...], xl[...], xt[...]], axis=-1)
    h = jnp.dot(x, w1[...], preferred_element_type=jnp.float32) + b1[...]
    h = jnp.maximum(h, 0.0)
    h = jnp.dot(h, w2[...], preferred_element_type=jnp.float32) + b2[...]
    h = jnp.maximum(h, 0.0)
    out[...] = jnp.dot(h, w3[...], preferred_element_type=jnp.float32) + b3[...]


def _mlp(xs, xl, xt, W1, b1, W2, b2, W3, b3):
    grid = BATCH // TILE
    emb_spec = pl.BlockSpec((TILE, EMBED), lambda i: (i, 0))
    full = lambda a: pl.BlockSpec(a.shape, lambda i: (0,) * a.ndim)
    return pl.pallas_call(
        _mlp_body,
        grid=(grid,),
        in_specs=[emb_spec, emb_spec,
                  emb_spec,
                  full(W1), full(b1), full(W2), full(b2), full(W3), full(b3)],
        out_specs=pl.BlockSpec((TILE, OUT), lambda i: (i, 0)),
        out_shape=jax.ShapeDtypeStruct((BATCH, OUT), jnp.float32),
    )(xs, xl, xt, W1, b1, W2, b2, W3, b3)


def kernel(service_idx, location_idx, time_idx, T_service, T_location,
           T_time, W1, b1, W2, b2, W3, b3):
    svc = service_idx.astype(jnp.int32)
    loc = location_idx.astype(jnp.int32)
    tim = time_idx.astype(jnp.int32)
    xs, xl, xt = _sc_gather(svc, loc, tim, T_service, T_location, T_time)
    return _mlp(xs, xl, xt, W1,
                b1.reshape(1, HIDDEN), W2, b2.reshape(1, HIDDEN),
                W3, b3.reshape(1, OUT))
